# baseline (speedup 1.0000x reference)
import functools

import jax
import jax.numpy as jnp
from jax import lax
from jax.experimental import pallas as pl
from jax.experimental.pallas import tpu as pltpu
from jax.experimental.pallas import tpu_sc as plsc

B = 1
S = 2048
D = 768
H = 12
DH = D // H
E = 64
K = 2
FF = 512
G = 256
CAP = 128

BT = 256
NBLK = S // BT
NW = 32
SLOTS = E * CAP
DUMMY = SLOTS


def _qkv_body(x_ref, g_ref, b_ref, wq_ref, wk_ref, wv_ref, q_ref, k_ref, v_ref):
    x = x_ref[...]
    mu = jnp.mean(x, axis=1, keepdims=True)
    var = jnp.mean((x - mu) ** 2, axis=1, keepdims=True)
    xn = (x - mu) * lax.rsqrt(var + 1e-5) * g_ref[...] + b_ref[...]
    scale = 1.0 / (DH ** 0.5)
    q_ref[...] = jnp.dot(xn, wq_ref[...], preferred_element_type=jnp.float32) * scale
    k_ref[...] = jnp.dot(xn, wk_ref[...], preferred_element_type=jnp.float32)
    v_ref[...] = jnp.dot(xn, wv_ref[...], preferred_element_type=jnp.float32)


def _qkv(x, g, b, wq, wk, wv):
    blk = pl.BlockSpec((BT, D), lambda i: (i, 0))
    full = pl.BlockSpec((D, D), lambda i: (0, 0))
    vec = pl.BlockSpec((1, D), lambda i: (0, 0))
    return pl.pallas_call(
        _qkv_body,
        grid=(NBLK,),
        in_specs=[blk, vec, vec, full, full, full],
        out_specs=[blk, blk, blk],
        out_shape=[jax.ShapeDtypeStruct((S, D), jnp.float32)] * 3,
    )(x, g, b, wq, wk, wv)


def _attn_body(q_ref, k_ref, v_ref, o_ref):
    qi = pl.program_id(1)
    q = q_ref[0]

    def body(kj, carry):
        acc, m, l = carry
        kblk = k_ref[0, pl.ds(kj * BT, BT), :]
        vblk = v_ref[0, pl.ds(kj * BT, BT), :]
        s = lax.dot_general(q, kblk, (((1,), (1,)), ((), ())),
                            preferred_element_type=jnp.float32)
        ri = lax.broadcasted_iota(jnp.int32, (BT, BT), 0) + qi * BT
        ci = lax.broadcasted_iota(jnp.int32, (BT, BT), 1) + kj * BT
        s = jnp.where(ri >= ci, s, -1e9)
        mnew = jnp.maximum(m, jnp.max(s, axis=1, keepdims=True))
        p = jnp.exp(s - mnew)
        alpha = jnp.exp(m - mnew)
        l = l * alpha + jnp.sum(p, axis=1, keepdims=True)
        acc = acc * alpha + jnp.dot(p, vblk, preferred_element_type=jnp.float32)
        return acc, mnew, l

    acc0 = jnp.zeros((BT, DH), jnp.float32)
    m0 = jnp.full((BT, 1), -1e30, jnp.float32)
    l0 = jnp.zeros((BT, 1), jnp.float32)
    acc, m, l = lax.fori_loop(0, qi + 1, body, (acc0, m0, l0))
    o_ref[0] = acc / l


def _attention(q, k, v):
    qspec = pl.BlockSpec((1, BT, DH), lambda h, qi: (h, qi, 0))
    kvspec = pl.BlockSpec((1, S, DH), lambda h, qi: (h, 0, 0))
    return pl.pallas_call(
        _attn_body,
        grid=(H, NBLK),
        in_specs=[qspec, kvspec, kvspec],
        out_specs=qspec,
        out_shape=jax.ShapeDtypeStruct((H, S, DH), jnp.float32),
    )(q, k, v)


def _post_body(x_ref, ctx_ref, wo_ref, g_ref, b_ref, wg1_ref, wg2_ref,
               h1_ref, x2_ref, lg_ref):
    h1 = x_ref[...] + jnp.dot(ctx_ref[...], wo_ref[...],
                              preferred_element_type=jnp.float32)
    h1_ref[...] = h1
    mu = jnp.mean(h1, axis=1, keepdims=True)
    var = jnp.mean((h1 - mu) ** 2, axis=1, keepdims=True)
    x2 = (h1 - mu) * lax.rsqrt(var + 1e-5) * g_ref[...] + b_ref[...]
    x2_ref[...] = x2
    t = jnp.tanh(jnp.dot(x2, wg1_ref[...], preferred_element_type=jnp.float32))
    lg_ref[...] = lax.dot_general(wg2_ref[...], t, (((0,), (1,)), ((), ())),
                                  preferred_element_type=jnp.float32)


def _post_attn(x, ctx, wo, g, b, wg1, wg2):
    blk = pl.BlockSpec((BT, D), lambda i: (i, 0))
    return pl.pallas_call(
        _post_body,
        grid=(NBLK,),
        in_specs=[blk, blk,
                  pl.BlockSpec((D, D), lambda i: (0, 0)),
                  pl.BlockSpec((1, D), lambda i: (0, 0)),
                  pl.BlockSpec((1, D), lambda i: (0, 0)),
                  pl.BlockSpec((D, G), lambda i: (0, 0)),
                  pl.BlockSpec((G, E), lambda i: (0, 0))],
        out_specs=[blk, blk, pl.BlockSpec((E, BT), lambda i: (0, i))],
        out_shape=[jax.ShapeDtypeStruct((S, D), jnp.float32),
                   jax.ShapeDtypeStruct((S, D), jnp.float32),
                   jax.ShapeDtypeStruct((E, S), jnp.float32)],
    )(x, ctx, wo, g, b, wg1, wg2)


def _route_body(lg_ref, s1_ref, s2_ref, c1_ref, c2_ref, w1_ref, w2_ref,
                carry_ref):
    i = pl.program_id(0)

    @pl.when(i == 0)
    def _():
        carry_ref[...] = jnp.zeros_like(carry_ref)

    l = lg_ref[...]
    ei = lax.broadcasted_iota(jnp.int32, (E, BT), 0)
    m1 = jnp.max(l, axis=0, keepdims=True)
    idx1 = jnp.min(jnp.where(l == m1, ei, E), axis=0, keepdims=True)
    oh1 = (ei == idx1)
    lm = jnp.where(oh1, -1e30, l)
    m2 = jnp.max(lm, axis=0, keepdims=True)
    idx2 = jnp.min(jnp.where(lm == m2, ei, E), axis=0, keepdims=True)
    oh2 = (ei == idx2)

    d = jnp.exp(m2 - m1)
    g1 = 1.0 / (1.0 + d)
    g2 = d / (1.0 + d)

    ohs = oh1.astype(jnp.float32) + oh2.astype(jnp.float32)
    rj = lax.broadcasted_iota(jnp.int32, (BT, BT), 0)
    cjj = lax.broadcasted_iota(jnp.int32, (BT, BT), 1)
    u = (rj < cjj).astype(jnp.float32)
    cnt = jnp.dot(ohs, u, preferred_element_type=jnp.float32)
    carry = carry_ref[:, 0:1]
    tot = carry + cnt
    pos1 = jnp.sum(oh1.astype(jnp.float32) * tot, axis=0, keepdims=True)
    pos2 = jnp.sum(oh2.astype(jnp.float32) * tot, axis=0, keepdims=True)
    carry_ref[:, 0:1] = carry + jnp.sum(ohs, axis=1, keepdims=True)

    keep1 = pos1 < CAP
    keep2 = pos2 < CAP
    p1 = jnp.minimum(pos1, CAP - 1).astype(jnp.int32)
    p2 = jnp.minimum(pos2, CAP - 1).astype(jnp.int32)
    comb1 = idx1 * CAP + p1
    comb2 = idx2 * CAP + p2
    s1_ref[...] = jnp.where(keep1, comb1, DUMMY).reshape(1, 1, BT)
    s2_ref[...] = jnp.where(keep2, comb2, DUMMY).reshape(1, 1, BT)
    c1_ref[...] = comb1.reshape(1, 1, BT)
    c2_ref[...] = comb2.reshape(1, 1, BT)
    w1_ref[...] = (g1 * keep1.astype(jnp.float32)).reshape(1, 1, BT)
    w2_ref[...] = (g2 * keep2.astype(jnp.float32)).reshape(1, 1, BT)


def _route(logitsT):
    iblk = pl.BlockSpec((1, 1, BT), lambda i: (i, 0, 0))
    ishape = jax.ShapeDtypeStruct((NBLK, 1, BT), jnp.int32)
    fshape = jax.ShapeDtypeStruct((NBLK, 1, BT), jnp.float32)
    return pl.pallas_call(
        _route_body,
        grid=(NBLK,),
        in_specs=[pl.BlockSpec((E, BT), lambda i: (0, i))],
        out_specs=[iblk] * 4 + [iblk] * 2,
        out_shape=[ishape, ishape, ishape, ishape, fshape, fshape],
        scratch_shapes=[pltpu.VMEM((E, 128), jnp.float32)],
    )(logitsT)


def _dispatch_body(s1_hbm, s2_hbm, x2_hbm, out_hbm,
                   s1_v, s2_v, sm0_v, sm1_v, rows_v, sem):
    wid = lax.axis_index("s") * 2 + lax.axis_index("c")
    base = wid * 2 * CAP
    pltpu.sync_copy(s1_hbm, s1_v)
    pltpu.sync_copy(s2_hbm, s2_v)
    for j in range(CAP // 16):
        z = jnp.zeros((16,), jnp.int32)
        sm0_v[pl.ds(j * 16, 16)] = z
        sm1_v[pl.ds(j * 16, 16)] = z

    def body(j, _):
        tok = j * 16 + lax.iota(jnp.int32, 16)
        for sv in (s1_v, s2_v):
            slot = sv[pl.ds(j * 16, 16)]
            r0 = slot - base
            msk0 = (r0 >= 0) & (r0 < CAP)
            plsc.store_scatter(sm0_v, [jnp.clip(r0, 0, CAP - 1)], tok, mask=msk0)
            r1 = r0 - CAP
            msk1 = (r1 >= 0) & (r1 < CAP)
            plsc.store_scatter(sm1_v, [jnp.clip(r1, 0, CAP - 1)], tok, mask=msk1)
        return 0

    lax.fori_loop(0, (K * S) // 16 // K, body, 0)
    pltpu.async_copy(x2_hbm.at[sm0_v], rows_v, sem).wait()
    pltpu.sync_copy(rows_v, out_hbm.at[pl.ds(base, CAP)])
    pltpu.async_copy(x2_hbm.at[sm1_v], rows_v, sem).wait()
    pltpu.sync_copy(rows_v, out_hbm.at[pl.ds(base + CAP, CAP)])


def _dispatch(s1, s2, x2):
    mesh = plsc.VectorSubcoreMesh(core_axis_name="c", subcore_axis_name="s")
    f = pl.kernel(
        _dispatch_body,
        mesh=mesh,
        compiler_params=pltpu.CompilerParams(needs_layout_passes=False),
        out_type=jax.ShapeDtypeStruct((SLOTS, D), jnp.float32),
        scratch_types=[
            pltpu.VMEM((S,), jnp.int32),
            pltpu.VMEM((S,), jnp.int32),
            pltpu.VMEM((CAP,), jnp.int32),
            pltpu.VMEM((CAP,), jnp.int32),
            pltpu.VMEM((CAP, D), jnp.float32),
            pltpu.SemaphoreType.DMA,
        ],
    )
    return f(s1, s2, x2)


def _ffn_body(xin_ref, w1_ref, w2_ref, out_ref):
    x = xin_ref[0]
    h = jax.nn.gelu(jnp.dot(x, w1_ref[0], preferred_element_type=jnp.float32))
    out_ref[0] = jnp.dot(h, w2_ref[0], preferred_element_type=jnp.float32)


def _ffn(expert_in, w1, w2):
    return pl.pallas_call(
        _ffn_body,
        grid=(E,),
        in_specs=[pl.BlockSpec((1, CAP, D), lambda e: (e, 0, 0)),
                  pl.BlockSpec((1, D, FF), lambda e: (e, 0, 0)),
                  pl.BlockSpec((1, FF, D), lambda e: (e, 0, 0))],
        out_specs=pl.BlockSpec((1, CAP, D), lambda e: (e, 0, 0)),
        out_shape=jax.ShapeDtypeStruct((E, CAP, D), jnp.float32),
    )(expert_in, w1, w2)


def _combine_body(cidx_hbm, eo_hbm, out_hbm, cidx_v, rows_v, sem):
    wid = lax.axis_index("s") * 2 + lax.axis_index("c")
    n = (K * S) // NW
    base = wid * n
    pltpu.sync_copy(cidx_hbm.at[pl.ds(base, n)], cidx_v)
    pltpu.async_copy(eo_hbm.at[cidx_v], rows_v, sem).wait()
    pltpu.sync_copy(rows_v, out_hbm.at[pl.ds(base, n)])


def _combine(cidx, eo):
    n = (K * S) // NW
    mesh = plsc.VectorSubcoreMesh(core_axis_name="c", subcore_axis_name="s")
    f = pl.kernel(
        _combine_body,
        mesh=mesh,
        out_type=jax.ShapeDtypeStruct((K * S, D), jnp.float32),
        scratch_types=[
            pltpu.VMEM((n,), jnp.int32),
            pltpu.VMEM((n, D), jnp.float32),
            pltpu.SemaphoreType.DMA,
        ],
    )
    return f(cidx, eo)


def _final_body(h1_ref, y0_ref, y1_ref, w1_ref, w2_ref, o_ref):
    o_ref[...] = (h1_ref[...] + w1_ref[...] * y0_ref[0] + w2_ref[...] * y1_ref[0])


def _final(h1, yrep, w1, w2):
    blk = pl.BlockSpec((BT, D), lambda i: (i, 0))
    return pl.pallas_call(
        _final_body,
        grid=(NBLK,),
        in_specs=[blk,
                  pl.BlockSpec((1, BT, D), lambda i: (0, i, 0)),
                  pl.BlockSpec((1, BT, D), lambda i: (1, i, 0)),
                  pl.BlockSpec((BT, 1), lambda i: (i, 0)),
                  pl.BlockSpec((BT, 1), lambda i: (i, 0))],
        out_specs=blk,
        out_shape=jax.ShapeDtypeStruct((S, D), jnp.float32),
    )(h1, yrep, yrep, w1, w2)


def kernel(hidden_states, ln1_g, ln1_b, Wq, Wk, Wv, Wo, ln2_g, ln2_b,
           Wg1, Wg2, W1, W2):
    x = hidden_states.reshape(S, D)
    q, k, v = _qkv(x, ln1_g.reshape(1, D), ln1_b.reshape(1, D), Wq, Wk, Wv)
    qh, kh, vh = (a.reshape(S, H, DH).transpose(1, 0, 2) for a in (q, k, v))
    ctx = _attention(qh, kh, vh).transpose(1, 0, 2).reshape(S, D)
    h1, x2, logitsT = _post_attn(x, ctx, Wo, ln2_g.reshape(1, D),
                                 ln2_b.reshape(1, D), Wg1, Wg2)
    s1, s2, c1, c2, w1, w2 = _route(logitsT)
    s1 = s1.reshape(S)
    s2 = s2.reshape(S)
    cidx = jnp.concatenate([c1.reshape(S), c2.reshape(S)])
    expert_in = _dispatch(s1, s2, x2)
    eo = _ffn(expert_in.reshape(E, CAP, D), W1, W2)
    yrep = _combine(cidx, eo.reshape(SLOTS, D))
    out = _final(h1, yrep.reshape(2, S, D), w1.reshape(S, 1), w2.reshape(S, 1))
    return out.reshape(B, S, D)
